# baseline (speedup 1.0000x reference)
import functools

import jax
import jax.numpy as jnp
from jax import lax
from jax.experimental import pallas as pl
from jax.experimental.pallas import tpu as pltpu
from jax.experimental.pallas import tpu_sc as plsc

_N = 16384
_D4 = 4
_NS = 1_000_000
_BINS = 50
_MBP = 2560
_NPAD = 1 << 20
_ROWS = _NPAD // 128
_RB = 1024
_BIGI = 2_000_000_000

_SUBS = 16
_WIN = 1_600_000
_NWIN = 4
_HISTP = _WIN * _NWIN
_GARB = 8192
_MOFF = _WIN + _GARB
_WBUF = _MOFF + _MBP
_CHUNK = 2048
_SHARD = _NPAD // _SUBS
_NCH = _SHARD // _CHUNK
_ZC = 10_000
_ZH = _ZC // 2
_WSLICE = _WIN // _SUBS


def _stats_body(q_ref, r_ref, mean_ref, cov_ref, linq_ref, linr_ref):
    q = q_ref[...]
    r = r_ref[...]
    x = jnp.concatenate([q, r], axis=1)
    mean = jnp.sum(x, axis=0, keepdims=True) * (1.0 / _N)
    c = x - mean
    cov = lax.dot_general(c, c, (((0,), (0,)), ((), ())),
                          preferred_element_type=jnp.float32,
                          precision=lax.Precision.HIGHEST)
    mean_ref[...] = mean
    cov_ref[...] = cov
    for src, lref in ((q, linq_ref), (r, linr_ref)):
        mn = jnp.min(src)
        mx = jnp.max(src)
        t = (src - mn) / (mx - mn) * jnp.float32(_BINS)
        idx = jnp.clip(jnp.floor(t).astype(jnp.int32), 0, _BINS - 1)
        lref[...] = idx[:, 0] * _BINS + idx[:, 1]


def _stats_call(q, r):
    return pl.pallas_call(
        _stats_body,
        out_shape=(
            jax.ShapeDtypeStruct((1, 4), jnp.float32),
            jax.ShapeDtypeStruct((4, 4), jnp.float32),
            jax.ShapeDtypeStruct((_N,), jnp.int32),
            jax.ShapeDtypeStruct((_N,), jnp.int32),
        ),
    )(q, r)


def _samples_from(z_ref, mean_ref, lt_ref, d):
    s = z_ref[0] * lt_ref[0, d]
    for k in range(1, _D4):
        s = s + z_ref[k] * lt_ref[k, d]
    return s + mean_ref[0, d]


def _minmax_body(z_ref, mean_ref, lt_ref, lo_ref, hi_ref):
    i = pl.program_id(0)
    lo = jnp.float32(jnp.inf)
    hi = jnp.float32(-jnp.inf)
    for d in range(_D4):
        s = _samples_from(z_ref, mean_ref, lt_ref, d)
        lo = jnp.minimum(lo, jnp.min(s))
        hi = jnp.maximum(hi, jnp.max(s))

    lo2 = jnp.reshape(lo, (1, 1))
    hi2 = jnp.reshape(hi, (1, 1))

    @pl.when(i == 0)
    def _():
        lo_ref[...] = lo2
        hi_ref[...] = hi2

    @pl.when(i != 0)
    def _():
        lo_ref[...] = jnp.minimum(lo_ref[...], lo2)
        hi_ref[...] = jnp.maximum(hi_ref[...], hi2)


def _minmax_call(z3, mean, lt):
    return pl.pallas_call(
        _minmax_body,
        grid=(_ROWS // _RB,),
        in_specs=[
            pl.BlockSpec((_D4, _RB, 128), lambda i: (0, i, 0)),
            pl.BlockSpec((1, 4), lambda i: (0, 0)),
            pl.BlockSpec((4, 4), lambda i: (0, 0)),
        ],
        out_specs=(
            pl.BlockSpec((1, 1), lambda i: (0, 0)),
            pl.BlockSpec((1, 1), lambda i: (0, 0)),
        ),
        out_shape=(
            jax.ShapeDtypeStruct((1, 1), jnp.float32),
            jax.ShapeDtypeStruct((1, 1), jnp.float32),
        ),
    )(z3, mean, lt)


def _bin_body(z_ref, mean_ref, lt_ref, lohi_ref, lin_ref):
    i = pl.program_id(0)
    lo = lohi_ref[0, 0]
    r50 = jnp.float32(_BINS) / (lohi_ref[0, 1] - lo)
    lin = None
    for d in range(_D4):
        s = _samples_from(z_ref, mean_ref, lt_ref, d)
        t = (s - lo) * r50
        idx = jnp.clip(t.astype(jnp.int32), 0, _BINS - 1)
        lin = idx if lin is None else lin * _BINS + idx
    rowi = lax.broadcasted_iota(jnp.int32, (_RB, 128), 0)
    lanei = lax.broadcasted_iota(jnp.int32, (_RB, 128), 1)
    flat = (i * _RB + rowi) * 128 + lanei
    lin_ref[...] = jnp.where(flat < _NS, lin, _BIGI + flat)


def _bin_call(z3, mean, lt, lohi):
    return pl.pallas_call(
        _bin_body,
        grid=(_ROWS // _RB,),
        in_specs=[
            pl.BlockSpec((_D4, _RB, 128), lambda i: (0, i, 0)),
            pl.BlockSpec((1, 4), lambda i: (0, 0)),
            pl.BlockSpec((4, 4), lambda i: (0, 0)),
            pl.BlockSpec((1, 2), lambda i: (0, 0)),
        ],
        out_specs=pl.BlockSpec((_RB, 128), lambda i: (i, 0)),
        out_shape=jax.ShapeDtypeStruct((_ROWS, 128), jnp.int32),
    )(z3, mean, lt, lohi)


def _sc_hist_call(lin, linq, linr):
    mesh = plsc.VectorSubcoreMesh(core_axis_name="c", subcore_axis_name="s")

    @functools.partial(
        pl.kernel,
        out_type=(
            jax.ShapeDtypeStruct((_HISTP,), jnp.float32),
            jax.ShapeDtypeStruct((_MBP,), jnp.float32),
            jax.ShapeDtypeStruct((_MBP,), jnp.float32),
        ),
        mesh=mesh,
        compiler_params=pltpu.CompilerParams(needs_layout_passes=False),
        scratch_types=[
            pltpu.VMEM_SHARED((_WBUF,), jnp.float32),
            pltpu.VMEM((_CHUNK,), jnp.int32),
            pltpu.VMEM((_CHUNK,), jnp.int32),
            pltpu.VMEM((2 * _CHUNK + 16,), jnp.int32),
            pltpu.VMEM((_CHUNK,), jnp.int32),
            pltpu.VMEM((_CHUNK,), jnp.float32),
            pltpu.VMEM((_ZC,), jnp.float32),
            pltpu.SemaphoreType.DMA,
            pltpu.SemaphoreType.DMA,
            pltpu.SemaphoreType.DMA,
            pltpu.SemaphoreType.DMA,
        ],
    )
    def sc_body(lin_hbm, linq_hbm, linr_hbm, hist_hbm, hq_hbm, hr_hbm,
                win, lbufa, lbufb, cbuf, fbuf, ones, stage,
                sema, semb, semz, semw):
        c = lax.axis_index("c")
        s = lax.axis_index("s")
        ones16 = jnp.ones((16,), jnp.float32)
        z16 = jnp.zeros((16,), jnp.float32)
        iota16 = lax.iota(jnp.int32, 16)
        wlim = jnp.uint32(_WIN)
        gmask = _GARB - 1

        @plsc.parallel_loop(0, _CHUNK // 16, unroll=8)
        def _(i):
            ones[pl.ds(i * 16, 16)] = ones16

        def fill_zeros():
            @plsc.parallel_loop(0, _ZC // 16, unroll=8)
            def _(i):
                stage[pl.ds(i * 16, 16)] = z16

        def cxform(src, base, woff):
            def body(i, woff):
                v = src[pl.ds(i * 16, 16)]
                u = plsc.bitcast(v - base, jnp.uint32)
                m = u < wlim
                plsc.store_compressed(cbuf.at[pl.ds(woff, 16)],
                                      plsc.bitcast(u, jnp.int32), mask=m)
                return woff + jnp.sum(m.astype(jnp.int32))

            return plsc.parallel_loop(
                0, _CHUNK // 16, unroll=4, carry=woff)(body)

        def flush(woff):
            @plsc.parallel_loop(0, _CHUNK // 16, unroll=8)
            def _(i):
                fbuf[pl.ds(i * 16, 16)] = cbuf[pl.ds(i * 16, 16)]

            pltpu.sync_copy(ones, win.at[fbuf], add=True)
            nmove = (woff - _CHUNK + 15) // 16

            def mv(i, _):
                cbuf[pl.ds(i * 16, 16)] = cbuf[pl.ds(_CHUNK + i * 16, 16)]
                return 0

            lax.fori_loop(0, nmove, mv, 0)
            return woff - _CHUNK

        def maybe_flush(woff):
            return lax.cond(woff >= _CHUNK, flush, lambda w: w, woff)

        def drain(woff):
            npad = (_CHUNK - woff + 15) // 16

            def pad(i, _):
                off = woff + i * 16
                cbuf[pl.ds(off, 16)] = _WIN + ((off + iota16) & gmask)
                return 0

            lax.fori_loop(0, npad, pad, 0)

            @plsc.parallel_loop(0, _CHUNK // 16, unroll=8)
            def _(i):
                fbuf[pl.ds(i * 16, 16)] = cbuf[pl.ds(i * 16, 16)]

            pltpu.sync_copy(ones, win.at[fbuf], add=True)

        def marg_scatter(src_hbm):
            for t in range(_N // _CHUNK):
                pltpu.sync_copy(src_hbm.at[pl.ds(t * _CHUNK, _CHUNK)], lbufa)

                @plsc.parallel_loop(0, _CHUNK // 16, unroll=8)
                def _(i):
                    fbuf[pl.ds(i * 16, 16)] = lbufa[pl.ds(i * 16, 16)] + _MOFF

                pltpu.sync_copy(ones, win.at[fbuf], add=True)

        def load_chunk(t, buf, sem):
            return pltpu.async_copy(
                lin_hbm.at[pl.ds(s * _SHARD + t * _CHUNK, _CHUNK)], buf, sem)

        def wait_load(buf, sem):
            pltpu.make_async_copy(
                lin_hbm.at[pl.ds(0, _CHUNK)], buf, sem).wait()

        for w in range(2):
            base = (2 * c + w) * _WIN
            fill_zeros()
            for t in range(_WSLICE // _ZC):
                pltpu.async_copy(
                    stage, win.at[pl.ds(s * _WSLICE + t * _ZC, _ZC)], semz)
            for t in range(_WSLICE // _ZC):
                pltpu.make_async_copy(
                    stage, win.at[pl.ds(0, _ZC)], semz).wait()
            if w == 0:
                @pl.when(s == 0)
                def _():
                    pltpu.sync_copy(stage.at[pl.ds(0, _MBP)],
                                    win.at[pl.ds(_MOFF, _MBP)])
            plsc.subcore_barrier()
            if w == 0:
                @pl.when(s == 0)
                def _():
                    @pl.when(c == 0)
                    def _():
                        marg_scatter(linq_hbm)

                    @pl.when(c == 1)
                    def _():
                        marg_scatter(linr_hbm)
            load_chunk(0, lbufa, sema)

            def pair(t, woff):
                wait_load(lbufa, sema)
                load_chunk(2 * t + 1, lbufb, semb)
                woff = maybe_flush(cxform(lbufa, base, woff))
                wait_load(lbufb, semb)

                @pl.when(t < _NCH // 2 - 1)
                def _():
                    load_chunk(2 * t + 2, lbufa, sema)

                return maybe_flush(cxform(lbufb, base, woff))

            woff = lax.fori_loop(0, _NCH // 2, pair, jnp.int32(0))
            drain(woff)
            plsc.subcore_barrier()
            for t in range(_WSLICE // _ZH):
                h = (t % 2) * _ZH
                if t >= 2:
                    pltpu.make_async_copy(
                        stage.at[pl.ds(h, _ZH)],
                        hist_hbm.at[pl.ds(0, _ZH)], semw).wait()
                off = s * _WSLICE + t * _ZH
                pltpu.sync_copy(win.at[pl.ds(off, _ZH)],
                                stage.at[pl.ds(h, _ZH)])
                pltpu.async_copy(stage.at[pl.ds(h, _ZH)],
                                 hist_hbm.at[pl.ds(base + off, _ZH)], semw)
            for t in range(2):
                pltpu.make_async_copy(
                    stage.at[pl.ds(0, _ZH)],
                    hist_hbm.at[pl.ds(0, _ZH)], semw).wait()
            if w == 0:
                @pl.when(s == 0)
                def _():
                    pltpu.sync_copy(win.at[pl.ds(_MOFF, _MBP)],
                                    stage.at[pl.ds(0, _MBP)])

                    @pl.when(c == 0)
                    def _():
                        pltpu.sync_copy(stage.at[pl.ds(0, _MBP)], hq_hbm)

                    @pl.when(c == 1)
                    def _():
                        pltpu.sync_copy(stage.at[pl.ds(0, _MBP)], hr_hbm)

    return sc_body(lin, linq, linr)


def _ent_body(h_ref, s1_ref, s2_ref):
    i = pl.program_id(0)
    h = h_ref[...]
    safe = jnp.where(h > 0, h, 1.0)
    clogc = jnp.sum(h * jnp.log(safe))
    tot = jnp.sum(h)

    c2 = jnp.reshape(clogc, (1, 1))
    t2 = jnp.reshape(tot, (1, 1))

    @pl.when(i == 0)
    def _():
        s1_ref[...] = c2
        s2_ref[...] = t2

    @pl.when(i != 0)
    def _():
        s1_ref[...] = s1_ref[...] + c2
        s2_ref[...] = s2_ref[...] + t2


def _ent_call(h2, blk):
    rows = h2.shape[0]
    return pl.pallas_call(
        _ent_body,
        grid=(rows // blk,),
        in_specs=[pl.BlockSpec((blk, 128), lambda i: (i, 0))],
        out_specs=(
            pl.BlockSpec((1, 1), lambda i: (0, 0)),
            pl.BlockSpec((1, 1), lambda i: (0, 0)),
        ),
        out_shape=(
            jax.ShapeDtypeStruct((1, 1), jnp.float32),
            jax.ShapeDtypeStruct((1, 1), jnp.float32),
        ),
    )(h2)


def _entropy(s1, s2):
    tot = s2[0, 0]
    return jnp.log(tot) - s1[0, 0] / tot


_Z3 = jnp.pad(
    jax.random.normal(jax.random.key(42), (_NS, _D4), dtype=jnp.float32).T,
    ((0, 0), (0, _NPAD - _NS))).reshape(_D4, _ROWS, 128)


def kernel(query_embedding, result_embedding):
    mean, cov_sum, linq, linr = _stats_call(query_embedding, result_embedding)
    cov = cov_sum / jnp.float32(_N - 1) + 1e-6 * jnp.eye(4, dtype=jnp.float32)
    lt = jnp.linalg.cholesky(cov).T

    z3 = _Z3

    lo, hi = _minmax_call(z3, mean, lt)
    lohi = jnp.concatenate([lo, hi], axis=1)
    lin = _bin_call(z3, mean, lt, lohi)

    hist, hq, hr = _sc_hist_call(lin.reshape(-1), linq, linr)

    sj1, sj2 = _ent_call(hist.reshape(_HISTP // 128, 128), 5000)
    sq1, sq2 = _ent_call(hq.reshape(_MBP // 128, 128), _MBP // 128)
    sr1, sr2 = _ent_call(hr.reshape(_MBP // 128, 128), _MBP // 128)

    joint_h = _entropy(sj1, sj2)
    max_h = _entropy(sq1, sq2) + _entropy(sr1, sr2)
    return jnp.clip(joint_h / max_h, 0.0, 1.0)


# speedup vs baseline: 1.0704x; 1.0704x over previous
import functools

import jax
import jax.numpy as jnp
from jax import lax
from jax.experimental import pallas as pl
from jax.experimental.pallas import tpu as pltpu
from jax.experimental.pallas import tpu_sc as plsc

_N = 16384
_D4 = 4
_NS = 1_000_000
_BINS = 50
_MBP = 2560
_NPAD = 1 << 20
_ROWS = _NPAD // 128
_RB = 1024
_BIGI = 2_000_000_000

_SUBS = 16
_WIN = 1_600_000
_NWIN = 4
_HISTP = _WIN * _NWIN
_GARB = 131072
_MOFF = _WIN + _GARB
_WBUF = _MOFF + _MBP
_CHUNK = 2048
_SHARD = _NPAD // _SUBS
_NCH = _SHARD // _CHUNK
_ZC = 10_000
_ZH = _ZC // 2
_WSLICE = _WIN // _SUBS


def _stats_body(q_ref, r_ref, mean_ref, cov_ref, linq_ref, linr_ref):
    q = q_ref[...]
    r = r_ref[...]
    x = jnp.concatenate([q, r], axis=1)
    mean = jnp.sum(x, axis=0, keepdims=True) * (1.0 / _N)
    c = x - mean
    cov = lax.dot_general(c, c, (((0,), (0,)), ((), ())),
                          preferred_element_type=jnp.float32,
                          precision=lax.Precision.HIGHEST)
    mean_ref[...] = mean
    cov_ref[...] = cov
    for src, lref in ((q, linq_ref), (r, linr_ref)):
        mn = jnp.min(src)
        mx = jnp.max(src)
        t = (src - mn) / (mx - mn) * jnp.float32(_BINS)
        idx = jnp.clip(jnp.floor(t).astype(jnp.int32), 0, _BINS - 1)
        lref[...] = idx[:, 0] * _BINS + idx[:, 1]


def _stats_call(q, r):
    return pl.pallas_call(
        _stats_body,
        out_shape=(
            jax.ShapeDtypeStruct((1, 4), jnp.float32),
            jax.ShapeDtypeStruct((4, 4), jnp.float32),
            jax.ShapeDtypeStruct((_N,), jnp.int32),
            jax.ShapeDtypeStruct((_N,), jnp.int32),
        ),
    )(q, r)


def _samples_from(z_ref, mean_ref, lt_ref, d):
    s = z_ref[0] * lt_ref[0, d]
    for k in range(1, _D4):
        s = s + z_ref[k] * lt_ref[k, d]
    return s + mean_ref[0, d]


def _minmax_body(z_ref, mean_ref, lt_ref, lo_ref, hi_ref):
    i = pl.program_id(0)
    lo = jnp.float32(jnp.inf)
    hi = jnp.float32(-jnp.inf)
    for d in range(_D4):
        s = _samples_from(z_ref, mean_ref, lt_ref, d)
        lo = jnp.minimum(lo, jnp.min(s))
        hi = jnp.maximum(hi, jnp.max(s))

    lo2 = jnp.reshape(lo, (1, 1))
    hi2 = jnp.reshape(hi, (1, 1))

    @pl.when(i == 0)
    def _():
        lo_ref[...] = lo2
        hi_ref[...] = hi2

    @pl.when(i != 0)
    def _():
        lo_ref[...] = jnp.minimum(lo_ref[...], lo2)
        hi_ref[...] = jnp.maximum(hi_ref[...], hi2)


def _minmax_call(z3, mean, lt):
    return pl.pallas_call(
        _minmax_body,
        grid=(_ROWS // _RB,),
        in_specs=[
            pl.BlockSpec((_D4, _RB, 128), lambda i: (0, i, 0)),
            pl.BlockSpec((1, 4), lambda i: (0, 0)),
            pl.BlockSpec((4, 4), lambda i: (0, 0)),
        ],
        out_specs=(
            pl.BlockSpec((1, 1), lambda i: (0, 0)),
            pl.BlockSpec((1, 1), lambda i: (0, 0)),
        ),
        out_shape=(
            jax.ShapeDtypeStruct((1, 1), jnp.float32),
            jax.ShapeDtypeStruct((1, 1), jnp.float32),
        ),
    )(z3, mean, lt)


def _bin_body(z_ref, mean_ref, lt_ref, lohi_ref, lin_ref):
    i = pl.program_id(0)
    lo = lohi_ref[0, 0]
    r50 = jnp.float32(_BINS) / (lohi_ref[0, 1] - lo)
    lin = None
    for d in range(_D4):
        s = _samples_from(z_ref, mean_ref, lt_ref, d)
        t = (s - lo) * r50
        idx = jnp.clip(t.astype(jnp.int32), 0, _BINS - 1)
        lin = idx if lin is None else lin * _BINS + idx
    rowi = lax.broadcasted_iota(jnp.int32, (_RB, 128), 0)
    lanei = lax.broadcasted_iota(jnp.int32, (_RB, 128), 1)
    flat = (i * _RB + rowi) * 128 + lanei
    lin_ref[...] = jnp.where(flat < _NS, lin, _BIGI + flat)


def _bin_call(z3, mean, lt, lohi):
    return pl.pallas_call(
        _bin_body,
        grid=(_ROWS // _RB,),
        in_specs=[
            pl.BlockSpec((_D4, _RB, 128), lambda i: (0, i, 0)),
            pl.BlockSpec((1, 4), lambda i: (0, 0)),
            pl.BlockSpec((4, 4), lambda i: (0, 0)),
            pl.BlockSpec((1, 2), lambda i: (0, 0)),
        ],
        out_specs=pl.BlockSpec((_RB, 128), lambda i: (i, 0)),
        out_shape=jax.ShapeDtypeStruct((_ROWS, 128), jnp.int32),
    )(z3, mean, lt, lohi)


def _sc_hist_call(lin, linq, linr):
    mesh = plsc.VectorSubcoreMesh(core_axis_name="c", subcore_axis_name="s")

    @functools.partial(
        pl.kernel,
        out_type=(
            jax.ShapeDtypeStruct((_HISTP,), jnp.float32),
            jax.ShapeDtypeStruct((_MBP,), jnp.float32),
            jax.ShapeDtypeStruct((_MBP,), jnp.float32),
        ),
        mesh=mesh,
        compiler_params=pltpu.CompilerParams(needs_layout_passes=False),
        scratch_types=[
            pltpu.VMEM_SHARED((_WBUF,), jnp.float32),
            pltpu.VMEM((_CHUNK,), jnp.int32),
            pltpu.VMEM((_CHUNK,), jnp.int32),
            pltpu.VMEM((_CHUNK,), jnp.int32),
            pltpu.VMEM((_CHUNK,), jnp.int32),
            pltpu.VMEM((_CHUNK,), jnp.float32),
            pltpu.VMEM((_ZC,), jnp.float32),
            pltpu.SemaphoreType.DMA,
            pltpu.SemaphoreType.DMA,
            pltpu.SemaphoreType.DMA,
            pltpu.SemaphoreType.DMA,
            pltpu.SemaphoreType.DMA,
            pltpu.SemaphoreType.DMA,
        ],
    )
    def sc_body(lin_hbm, linq_hbm, linr_hbm, hist_hbm, hq_hbm, hr_hbm,
                win, lbufa, lbufb, ibufa, ibufb, ones, stage,
                sema, semb, semc, semd, semz, semw):
        c = lax.axis_index("c")
        s = lax.axis_index("s")
        ones16 = jnp.ones((16,), jnp.float32)
        z16 = jnp.zeros((16,), jnp.float32)
        wlim = jnp.uint32(_WIN)
        gmask = jnp.uint32(_GARB - 1)

        @plsc.parallel_loop(0, _CHUNK // 16, unroll=8)
        def _(i):
            ones[pl.ds(i * 16, 16)] = ones16

        def fill_zeros():
            @plsc.parallel_loop(0, _ZC // 16, unroll=8)
            def _(i):
                stage[pl.ds(i * 16, 16)] = z16

        def xform(src, dst, base):
            @plsc.parallel_loop(0, _CHUNK // 16, unroll=8)
            def _(i):
                v = src[pl.ds(i * 16, 16)]
                u = plsc.bitcast(v - base, jnp.uint32)
                g = wlim + (u & gmask)
                dst[pl.ds(i * 16, 16)] = plsc.bitcast(
                    jnp.minimum(u, g), jnp.int32)

        def marg_scatter(src_hbm):
            for t in range(_N // _CHUNK):
                pltpu.sync_copy(src_hbm.at[pl.ds(t * _CHUNK, _CHUNK)], lbufa)

                @plsc.parallel_loop(0, _CHUNK // 16, unroll=8)
                def _(i):
                    ibufa[pl.ds(i * 16, 16)] = lbufa[pl.ds(i * 16, 16)] + _MOFF

                pltpu.sync_copy(ones, win.at[ibufa], add=True)

        def load_chunk(t, buf, sem):
            return pltpu.async_copy(
                lin_hbm.at[pl.ds(s * _SHARD + t * _CHUNK, _CHUNK)], buf, sem)

        def wait_load(buf, sem):
            pltpu.make_async_copy(
                lin_hbm.at[pl.ds(0, _CHUNK)], buf, sem).wait()

        for w in range(2):
            base = (2 * c + w) * _WIN
            fill_zeros()
            for t in range(_WSLICE // _ZC):
                pltpu.async_copy(
                    stage, win.at[pl.ds(s * _WSLICE + t * _ZC, _ZC)], semz)
            for t in range(_WSLICE // _ZC):
                pltpu.make_async_copy(
                    stage, win.at[pl.ds(0, _ZC)], semz).wait()
            if w == 0:
                @pl.when(s == 0)
                def _():
                    pltpu.sync_copy(stage.at[pl.ds(0, _MBP)],
                                    win.at[pl.ds(_MOFF, _MBP)])
            plsc.subcore_barrier()
            if w == 0:
                @pl.when(s == 0)
                def _():
                    @pl.when(c == 0)
                    def _():
                        marg_scatter(linq_hbm)

                    @pl.when(c == 1)
                    def _():
                        marg_scatter(linr_hbm)
            load_chunk(0, lbufa, sema)

            def wait_scatter(sem):
                pltpu.make_async_copy(ones, win.at[ibufa], sem).wait()

            def pair(t, _):
                wait_load(lbufa, sema)
                load_chunk(2 * t + 1, lbufb, semb)

                @pl.when(t > 0)
                def _():
                    wait_scatter(semc)

                xform(lbufa, ibufa, base)
                pltpu.async_copy(ones, win.at[ibufa], semc, add=True)
                wait_load(lbufb, semb)

                @pl.when(t < _NCH // 2 - 1)
                def _():
                    load_chunk(2 * t + 2, lbufa, sema)

                @pl.when(t > 0)
                def _():
                    wait_scatter(semd)

                xform(lbufb, ibufb, base)
                pltpu.async_copy(ones, win.at[ibufb], semd, add=True)
                return 0

            lax.fori_loop(0, _NCH // 2, pair, 0)
            wait_scatter(semc)
            wait_scatter(semd)
            plsc.subcore_barrier()
            for t in range(_WSLICE // _ZH):
                h = (t % 2) * _ZH
                if t >= 2:
                    pltpu.make_async_copy(
                        stage.at[pl.ds(h, _ZH)],
                        hist_hbm.at[pl.ds(0, _ZH)], semw).wait()
                off = s * _WSLICE + t * _ZH
                pltpu.sync_copy(win.at[pl.ds(off, _ZH)],
                                stage.at[pl.ds(h, _ZH)])
                pltpu.async_copy(stage.at[pl.ds(h, _ZH)],
                                 hist_hbm.at[pl.ds(base + off, _ZH)], semw)
            for t in range(2):
                pltpu.make_async_copy(
                    stage.at[pl.ds(0, _ZH)],
                    hist_hbm.at[pl.ds(0, _ZH)], semw).wait()
            if w == 0:
                @pl.when(s == 0)
                def _():
                    pltpu.sync_copy(win.at[pl.ds(_MOFF, _MBP)],
                                    stage.at[pl.ds(0, _MBP)])

                    @pl.when(c == 0)
                    def _():
                        pltpu.sync_copy(stage.at[pl.ds(0, _MBP)], hq_hbm)

                    @pl.when(c == 1)
                    def _():
                        pltpu.sync_copy(stage.at[pl.ds(0, _MBP)], hr_hbm)

    return sc_body(lin, linq, linr)


def _ent_body(h_ref, s1_ref, s2_ref):
    i = pl.program_id(0)
    h = h_ref[...]
    safe = jnp.where(h > 0, h, 1.0)
    clogc = jnp.sum(h * jnp.log(safe))
    tot = jnp.sum(h)

    c2 = jnp.reshape(clogc, (1, 1))
    t2 = jnp.reshape(tot, (1, 1))

    @pl.when(i == 0)
    def _():
        s1_ref[...] = c2
        s2_ref[...] = t2

    @pl.when(i != 0)
    def _():
        s1_ref[...] = s1_ref[...] + c2
        s2_ref[...] = s2_ref[...] + t2


def _ent_call(h2, blk):
    rows = h2.shape[0]
    return pl.pallas_call(
        _ent_body,
        grid=(rows // blk,),
        in_specs=[pl.BlockSpec((blk, 128), lambda i: (i, 0))],
        out_specs=(
            pl.BlockSpec((1, 1), lambda i: (0, 0)),
            pl.BlockSpec((1, 1), lambda i: (0, 0)),
        ),
        out_shape=(
            jax.ShapeDtypeStruct((1, 1), jnp.float32),
            jax.ShapeDtypeStruct((1, 1), jnp.float32),
        ),
    )(h2)


def _entropy(s1, s2):
    tot = s2[0, 0]
    return jnp.log(tot) - s1[0, 0] / tot


_Z3 = jnp.pad(
    jax.random.normal(jax.random.key(42), (_NS, _D4), dtype=jnp.float32).T,
    ((0, 0), (0, _NPAD - _NS))).reshape(_D4, _ROWS, 128)


def kernel(query_embedding, result_embedding):
    mean, cov_sum, linq, linr = _stats_call(query_embedding, result_embedding)
    cov = cov_sum / jnp.float32(_N - 1) + 1e-6 * jnp.eye(4, dtype=jnp.float32)
    lt = jnp.linalg.cholesky(cov).T

    z3 = _Z3

    lo, hi = _minmax_call(z3, mean, lt)
    lohi = jnp.concatenate([lo, hi], axis=1)
    lin = _bin_call(z3, mean, lt, lohi)

    hist, hq, hr = _sc_hist_call(lin.reshape(-1), linq, linr)

    sj1, sj2 = _ent_call(hist.reshape(_HISTP // 128, 128), 5000)
    sm1, _ = _ent_call(
        jnp.concatenate([hq, hr]).reshape(2 * _MBP // 128, 128),
        2 * _MBP // 128)

    joint_h = _entropy(sj1, sj2)
    max_h = 2.0 * jnp.log(jnp.float32(_N)) - sm1[0, 0] / jnp.float32(_N)
    return jnp.clip(joint_h / max_h, 0.0, 1.0)
